# Initial kernel scaffold; baseline (speedup 1.0000x reference)
import jax
import jax.numpy as jnp
from jax.experimental import pallas as pl


def kernel(X_n, nadj, edge_name, T, eadj, W1, b1, W2, b2, W6, b6, W3, b3, W4, b4, Wl1, bl1, Wl2, bl2):
    raise NotImplementedError("write your pallas kernel here")


# speedup vs baseline: 1.4241x; 1.4241x over previous
import functools

import jax
import jax.numpy as jnp
from jax import lax
from jax.experimental import pallas as pl
from jax.experimental.pallas import tpu as pltpu
from jax.experimental.pallas import tpu_sc as plsc

_N = 4096
_E = 8192
_NH = 64
_NC = 16

_F32 = jnp.float32


def _mm_body(x_ref, w_ref, o_ref):
    o_ref[...] = jnp.dot(x_ref[...], w_ref[...], preferred_element_type=_F32)


def _mm(x, w):
    m, k = x.shape
    _, n = w.shape
    return pl.pallas_call(
        _mm_body,
        out_shape=jax.ShapeDtypeStruct((m, n), _F32),
    )(x, w)


_BM = 512


def _pass1_body(a_ref, s_ref, b1_ref, b2_ref, x1_ref, d_ref):
    y = jnp.dot(a_ref[...], s_ref[...], preferred_element_type=_F32)
    x1 = y[:, :_NH] + b1_ref[...]
    x2 = y[:, _NH:] + b2_ref[...]
    x1_ref[...] = jnp.concatenate([x1, jnp.zeros_like(x1)], axis=1)
    d_ref[...] = x1 * x2 + 2.0 * x1


def _pass1(nadj, s, b1, b2):
    return pl.pallas_call(
        _pass1_body,
        grid=(_N // _BM,),
        in_specs=[
            pl.BlockSpec((_BM, _N), lambda i: (i, 0)),
            pl.BlockSpec((_N, 2 * _NH), lambda i: (0, 0)),
            pl.BlockSpec((1, _NH), lambda i: (0, 0)),
            pl.BlockSpec((1, _NH), lambda i: (0, 0)),
        ],
        out_specs=[
            pl.BlockSpec((_BM, 2 * _NH), lambda i: (i, 0)),
            pl.BlockSpec((_BM, _NH), lambda i: (i, 0)),
        ],
        out_shape=[jax.ShapeDtypeStruct((_N, 2 * _NH), _F32),
                   jax.ShapeDtypeStruct((_N, _NH), _F32)],
        compiler_params=pltpu.CompilerParams(
            dimension_semantics=("arbitrary",)),
    )(nadj, s, b1.reshape(1, _NH), b2.reshape(1, _NH))


_SC_CORES = 2
_SC_SUBCORES = 16
_NW = _SC_CORES * _SC_SUBCORES
_EPW = _E // _NW
_CHUNK = 128
_NCHUNK = _EPW // _CHUNK


def _gather_body(x1_hbm, e0_hbm, e1_hbm, out_hbm, i0_v, i1_v, r0_v, r1_v, sem):
    wid = lax.axis_index("s") * _SC_CORES + lax.axis_index("c")
    row = wid * _NCHUNK
    pltpu.sync_copy(e0_hbm.at[pl.ds(row, _NCHUNK)], i0_v)
    pltpu.sync_copy(e1_hbm.at[pl.ds(row, _NCHUNK)], i1_v)
    for c in range(_NCHUNK):
        pltpu.async_copy(
            x1_hbm.at[i0_v.at[c]], r0_v.at[pl.ds(c * _CHUNK, _CHUNK)], sem
        ).wait()
        pltpu.async_copy(
            x1_hbm.at[i1_v.at[c]], r1_v.at[pl.ds(c * _CHUNK, _CHUNK)], sem
        ).wait()

    def body(r, carry):
        for j in range(_NH // 16):
            a = r0_v[r, pl.ds(j * 16, 16)]
            b = r1_v[r, pl.ds(j * 16, 16)]
            r0_v[r, pl.ds(j * 16, 16)] = jnp.maximum(a * b, 0.0)
        return carry

    lax.fori_loop(0, _EPW, body, 0)
    pltpu.sync_copy(r0_v, out_hbm.at[pl.ds(wid * _EPW, _EPW)])


def _edge_gather(x1p, e0, e1):
    mesh = plsc.VectorSubcoreMesh(core_axis_name="c", subcore_axis_name="s")
    f = functools.partial(
        pl.kernel,
        mesh=mesh,
        out_type=jax.ShapeDtypeStruct((_E, 2 * _NH), _F32),
        scratch_types=[
            pltpu.VMEM((_NCHUNK, _CHUNK), jnp.int32),
            pltpu.VMEM((_NCHUNK, _CHUNK), jnp.int32),
            pltpu.VMEM((_EPW, 2 * _NH), _F32),
            pltpu.VMEM((_EPW, 2 * _NH), _F32),
            pltpu.SemaphoreType.DMA,
        ],
    )(_gather_body)
    return f(x1p, e0, e1)


def _eadj_body(a_ref, u_ref, b_ref, o_ref):
    y = jnp.dot(a_ref[...], u_ref[...], preferred_element_type=_F32)
    o_ref[...] = jnp.maximum(y + b_ref[...], 0.0)


def _eadj_pass(eadj, u, b6):
    return pl.pallas_call(
        _eadj_body,
        grid=(_E // _BM,),
        in_specs=[
            pl.BlockSpec((_BM, _E), lambda i: (i, 0)),
            pl.BlockSpec((_E, _NH), lambda i: (0, 0)),
            pl.BlockSpec((1, _NH), lambda i: (0, 0)),
        ],
        out_specs=pl.BlockSpec((_BM, _NH), lambda i: (i, 0)),
        out_shape=jax.ShapeDtypeStruct((_E, _NH), _F32),
        compiler_params=pltpu.CompilerParams(
            dimension_semantics=("arbitrary",)),
    )(eadj, u, b6.reshape(1, _NH))


def _tpass_body(t_ref, v_ref, d_ref, x1_ref, wl1_ref, bl1_ref, wl2_ref,
                bl2_ref, wa_ref, wb_ref, wc_ref, g_ref):
    y = jnp.dot(t_ref[...], v_ref[...], preferred_element_type=_F32)
    x3 = (jnp.dot(y[:, :_NH], wl1_ref[...], preferred_element_type=_F32)
          + bl1_ref[...] + d_ref[...])
    xe3 = (jnp.dot(y[:, _NH:], wl2_ref[...], preferred_element_type=_F32)
           + bl2_ref[...])
    x1 = x1_ref[:, :_NH]
    g_ref[...] = (
        jnp.dot(x3, wa_ref[...], preferred_element_type=_F32)
        + jnp.dot(x1, wb_ref[...], preferred_element_type=_F32)
        + jnp.dot(xe3, wc_ref[...], preferred_element_type=_F32))


def _t_pass(t, v, d, x1, wl1, bl1, wl2, bl2, wa, wb, wc):
    ng = 3 * _NC
    return pl.pallas_call(
        _tpass_body,
        grid=(_N // _BM,),
        in_specs=[
            pl.BlockSpec((_BM, _E), lambda i: (i, 0)),
            pl.BlockSpec((_E, 2 * _NH), lambda i: (0, 0)),
            pl.BlockSpec((_BM, _NH), lambda i: (i, 0)),
            pl.BlockSpec((_BM, 2 * _NH), lambda i: (i, 0)),
            pl.BlockSpec((_NH, _NH), lambda i: (0, 0)),
            pl.BlockSpec((1, _NH), lambda i: (0, 0)),
            pl.BlockSpec((_NH, _NH), lambda i: (0, 0)),
            pl.BlockSpec((1, _NH), lambda i: (0, 0)),
            pl.BlockSpec((_NH, ng), lambda i: (0, 0)),
            pl.BlockSpec((_NH, ng), lambda i: (0, 0)),
            pl.BlockSpec((_NH, ng), lambda i: (0, 0)),
        ],
        out_specs=pl.BlockSpec((_BM, ng), lambda i: (i, 0)),
        out_shape=jax.ShapeDtypeStruct((_N, ng), _F32),
        compiler_params=pltpu.CompilerParams(
            dimension_semantics=("arbitrary",)),
    )(t, v, d, x1, wl1, bl1.reshape(1, _NH), wl2, bl2.reshape(1, _NH),
      wa, wb, wc)


def _final_body(a_ref, g_ref, b_ref, o1_ref, o2_ref, o3_ref):
    y = jnp.dot(a_ref[...], g_ref[...], preferred_element_type=_F32)
    y = y + b_ref[...]
    for o_ref, lo in ((o1_ref, 0), (o2_ref, _NC), (o3_ref, 2 * _NC)):
        o = y[:, lo:lo + _NC]
        m = jnp.max(o, axis=1, keepdims=True)
        ls = jnp.log(jnp.sum(jnp.exp(o - m), axis=1, keepdims=True)) + m
        o_ref[...] = o - ls


def _final_pass(nadj, g, bcat):
    ng = 3 * _NC
    return pl.pallas_call(
        _final_body,
        grid=(_N // _BM,),
        in_specs=[
            pl.BlockSpec((_BM, _N), lambda i: (i, 0)),
            pl.BlockSpec((_N, ng), lambda i: (0, 0)),
            pl.BlockSpec((1, ng), lambda i: (0, 0)),
        ],
        out_specs=[pl.BlockSpec((_BM, _NC), lambda i: (i, 0))] * 3,
        out_shape=[jax.ShapeDtypeStruct((_N, _NC), _F32)] * 3,
        compiler_params=pltpu.CompilerParams(
            dimension_semantics=("arbitrary",)),
    )(nadj, g, bcat.reshape(1, ng))


def kernel(X_n, nadj, edge_name, T, eadj, W1, b1, W2, b2, W6, b6, W3, b3,
           W4, b4, Wl1, bl1, Wl2, bl2):
    e0 = edge_name[:, 0].reshape(_E // _CHUNK, _CHUNK)
    e1 = edge_name[:, 1].reshape(_E // _CHUNK, _CHUNK)

    s = _mm(X_n, jnp.concatenate([W1, W2], axis=1))
    x1p, d = _pass1(nadj, s, b1, b2)
    x_e_pad = _edge_gather(x1p, e0, e1)
    w6p = jnp.concatenate([W6, jnp.zeros((_NH, _NH), _F32)], axis=0)
    u = _mm(x_e_pad, w6p)
    x_e2 = _eadj_pass(eadj, u, b6)
    v = jnp.concatenate([x_e_pad[:, :_NH], x_e2], axis=1)
    zeros = jnp.zeros((_NH, _NC), _F32)
    wa = jnp.concatenate([W3[:_NH], zeros, zeros], axis=1)
    wb = jnp.concatenate([W3[_NH:2 * _NH], W4, zeros], axis=1)
    wc = jnp.concatenate([W3[2 * _NH:], zeros, W4], axis=1)
    g = _t_pass(T, v, d, x1p, Wl1, bl1, Wl2, bl2, wa, wb, wc)
    bcat = jnp.concatenate([b3, b4, b4])
    return _final_pass(nadj, g, bcat)


# speedup vs baseline: 1.4979x; 1.0518x over previous
import functools

import jax
import jax.numpy as jnp
from jax import lax
from jax.experimental import pallas as pl
from jax.experimental.pallas import tpu as pltpu
from jax.experimental.pallas import tpu_sc as plsc

_N = 4096
_E = 8192
_NH = 64
_NC = 16

_F32 = jnp.float32
_BF16 = jnp.bfloat16


def _bf(x):
    return x.astype(_BF16)


_BMN = 1024
_BME = 512


def _pass1_body(xn_ref, wc_ref, a_ref, b1_ref, b2_ref, x1_ref, d_ref, s_ref):
    @pl.when(pl.program_id(0) == 0)
    def _():
        s_ref[...] = _bf(jnp.dot(xn_ref[...], wc_ref[...],
                                 preferred_element_type=_F32))

    y = jnp.dot(_bf(a_ref[...]), s_ref[...], preferred_element_type=_F32)
    x1 = y[:, :_NH] + b1_ref[...]
    x2 = y[:, _NH:] + b2_ref[...]
    x1_ref[...] = jnp.concatenate([x1, jnp.zeros_like(x1)], axis=1)
    d_ref[...] = x1 * x2 + 2.0 * x1


def _pass1(x_n, wc, nadj, b1, b2):
    return pl.pallas_call(
        _pass1_body,
        grid=(_N // _BMN,),
        in_specs=[
            pl.BlockSpec((_N, 2 * _NH), lambda i: (0, 0)),
            pl.BlockSpec((2 * _NH, 2 * _NH), lambda i: (0, 0)),
            pl.BlockSpec((_BMN, _N), lambda i: (i, 0)),
            pl.BlockSpec((1, _NH), lambda i: (0, 0)),
            pl.BlockSpec((1, _NH), lambda i: (0, 0)),
        ],
        out_specs=[
            pl.BlockSpec((_BMN, 2 * _NH), lambda i: (i, 0)),
            pl.BlockSpec((_BMN, _NH), lambda i: (i, 0)),
        ],
        out_shape=[jax.ShapeDtypeStruct((_N, 2 * _NH), _F32),
                   jax.ShapeDtypeStruct((_N, _NH), _F32)],
        scratch_shapes=[pltpu.VMEM((_N, 2 * _NH), _BF16)],
        compiler_params=pltpu.CompilerParams(
            dimension_semantics=("arbitrary",)),
    )(x_n, wc, nadj, b1.reshape(1, _NH), b2.reshape(1, _NH))


_SC_CORES = 2
_SC_SUBCORES = 16
_NW = _SC_CORES * _SC_SUBCORES
_EPW = _E // _NW
_CHUNK = 128
_NCHUNK = _EPW // _CHUNK


def _gather_body(x1_hbm, e0_hbm, e1_hbm, out_hbm, i0_v, i1_v, r0_v, r1_v, sem):
    wid = lax.axis_index("s") * _SC_CORES + lax.axis_index("c")
    row = wid * _NCHUNK
    pltpu.sync_copy(e0_hbm.at[pl.ds(row, _NCHUNK)], i0_v)
    pltpu.sync_copy(e1_hbm.at[pl.ds(row, _NCHUNK)], i1_v)
    copies = []
    for c in range(_NCHUNK):
        copies.append(pltpu.async_copy(
            x1_hbm.at[i0_v.at[c]], r0_v.at[pl.ds(c * _CHUNK, _CHUNK)], sem))
        copies.append(pltpu.async_copy(
            x1_hbm.at[i1_v.at[c]], r1_v.at[pl.ds(c * _CHUNK, _CHUNK)], sem))
    for cp in copies:
        cp.wait()

    def body(r, carry):
        for j in range(_NH // 16):
            a = r0_v[r, pl.ds(j * 16, 16)]
            b = r1_v[r, pl.ds(j * 16, 16)]
            r0_v[r, pl.ds(j * 16, 16)] = jnp.maximum(a * b, 0.0)
        return carry

    lax.fori_loop(0, _EPW, body, 0)
    pltpu.sync_copy(r0_v, out_hbm.at[pl.ds(wid * _EPW, _EPW)])


def _edge_gather(x1p, e0, e1):
    mesh = plsc.VectorSubcoreMesh(core_axis_name="c", subcore_axis_name="s")
    f = functools.partial(
        pl.kernel,
        mesh=mesh,
        out_type=jax.ShapeDtypeStruct((_E, 2 * _NH), _F32),
        scratch_types=[
            pltpu.VMEM((_NCHUNK, _CHUNK), jnp.int32),
            pltpu.VMEM((_NCHUNK, _CHUNK), jnp.int32),
            pltpu.VMEM((_EPW, 2 * _NH), _F32),
            pltpu.VMEM((_EPW, 2 * _NH), _F32),
            pltpu.SemaphoreType.DMA,
        ],
    )(_gather_body)
    return f(x1p, e0, e1)


def _eadj_body(xep_ref, w6p_ref, a_ref, b_ref, o_ref, u_ref):
    @pl.when(pl.program_id(0) == 0)
    def _():
        u_ref[...] = _bf(jnp.dot(xep_ref[...], w6p_ref[...],
                                 preferred_element_type=_F32))

    y = jnp.dot(_bf(a_ref[...]), u_ref[...], preferred_element_type=_F32)
    o_ref[...] = jnp.maximum(y + b_ref[...], 0.0)


def _eadj_pass(x_e_pad, w6p, eadj, b6):
    return pl.pallas_call(
        _eadj_body,
        grid=(_E // _BME,),
        in_specs=[
            pl.BlockSpec((_E, 2 * _NH), lambda i: (0, 0)),
            pl.BlockSpec((2 * _NH, _NH), lambda i: (0, 0)),
            pl.BlockSpec((_BME, _E), lambda i: (i, 0)),
            pl.BlockSpec((1, _NH), lambda i: (0, 0)),
        ],
        out_specs=pl.BlockSpec((_BME, _NH), lambda i: (i, 0)),
        out_shape=jax.ShapeDtypeStruct((_E, _NH), _F32),
        scratch_shapes=[pltpu.VMEM((_E, _NH), _BF16)],
        compiler_params=pltpu.CompilerParams(
            dimension_semantics=("arbitrary",)),
    )(x_e_pad, w6p, eadj, b6.reshape(1, _NH))


def _tpass_body(t_ref, v_ref, d_ref, x1_ref, wl1_ref, bl1_ref, wl2_ref,
                bl2_ref, wa_ref, wb_ref, wc_ref, g_ref):
    y = jnp.dot(_bf(t_ref[...]), v_ref[...], preferred_element_type=_F32)
    x3 = (jnp.dot(y[:, :_NH], wl1_ref[...], preferred_element_type=_F32)
          + bl1_ref[...] + d_ref[...])
    xe3 = (jnp.dot(y[:, _NH:], wl2_ref[...], preferred_element_type=_F32)
           + bl2_ref[...])
    x1 = x1_ref[:, :_NH]
    g_ref[...] = _bf(
        jnp.dot(x3, wa_ref[...], preferred_element_type=_F32)
        + jnp.dot(x1, wb_ref[...], preferred_element_type=_F32)
        + jnp.dot(xe3, wc_ref[...], preferred_element_type=_F32))


def _t_pass(t, v, d, x1p, wl1, bl1, wl2, bl2, wa, wb, wc):
    ng = 3 * _NC
    return pl.pallas_call(
        _tpass_body,
        grid=(_N // _BME,),
        in_specs=[
            pl.BlockSpec((_BME, _E), lambda i: (i, 0)),
            pl.BlockSpec((_E, 2 * _NH), lambda i: (0, 0)),
            pl.BlockSpec((_BME, _NH), lambda i: (i, 0)),
            pl.BlockSpec((_BME, 2 * _NH), lambda i: (i, 0)),
            pl.BlockSpec((_NH, _NH), lambda i: (0, 0)),
            pl.BlockSpec((1, _NH), lambda i: (0, 0)),
            pl.BlockSpec((_NH, _NH), lambda i: (0, 0)),
            pl.BlockSpec((1, _NH), lambda i: (0, 0)),
            pl.BlockSpec((_NH, ng), lambda i: (0, 0)),
            pl.BlockSpec((_NH, ng), lambda i: (0, 0)),
            pl.BlockSpec((_NH, ng), lambda i: (0, 0)),
        ],
        out_specs=pl.BlockSpec((_BME, ng), lambda i: (i, 0)),
        out_shape=jax.ShapeDtypeStruct((_N, ng), _BF16),
        compiler_params=pltpu.CompilerParams(
            dimension_semantics=("arbitrary",)),
    )(t, v, d, x1p, wl1, bl1.reshape(1, _NH), wl2, bl2.reshape(1, _NH),
      wa, wb, wc)


def _final_body(a_ref, g_ref, b_ref, o1_ref, o2_ref, o3_ref):
    y = jnp.dot(_bf(a_ref[...]), g_ref[...], preferred_element_type=_F32)
    y = y + b_ref[...]
    for o_ref, lo in ((o1_ref, 0), (o2_ref, _NC), (o3_ref, 2 * _NC)):
        o = y[:, lo:lo + _NC]
        m = jnp.max(o, axis=1, keepdims=True)
        ls = jnp.log(jnp.sum(jnp.exp(o - m), axis=1, keepdims=True)) + m
        o_ref[...] = o - ls


def _final_pass(nadj, g, bcat):
    ng = 3 * _NC
    return pl.pallas_call(
        _final_body,
        grid=(_N // _BMN,),
        in_specs=[
            pl.BlockSpec((_BMN, _N), lambda i: (i, 0)),
            pl.BlockSpec((_N, ng), lambda i: (0, 0)),
            pl.BlockSpec((1, ng), lambda i: (0, 0)),
        ],
        out_specs=[pl.BlockSpec((_BMN, _NC), lambda i: (i, 0))] * 3,
        out_shape=[jax.ShapeDtypeStruct((_N, _NC), _F32)] * 3,
        compiler_params=pltpu.CompilerParams(
            dimension_semantics=("arbitrary",)),
    )(nadj, g, bcat.reshape(1, ng))


def kernel(X_n, nadj, edge_name, T, eadj, W1, b1, W2, b2, W6, b6, W3, b3,
           W4, b4, Wl1, bl1, Wl2, bl2):
    e0 = edge_name[:, 0].reshape(_E // _CHUNK, _CHUNK)
    e1 = edge_name[:, 1].reshape(_E // _CHUNK, _CHUNK)

    wc = jnp.concatenate([W1, W2], axis=1)
    x1p, d = _pass1(X_n, wc, nadj, b1, b2)
    x_e_pad = _edge_gather(x1p, e0, e1)
    w6p = jnp.concatenate([W6, jnp.zeros((_NH, _NH), _F32)], axis=0)
    x_e2 = _eadj_pass(x_e_pad, w6p, eadj, b6)
    v = _bf(jnp.concatenate([x_e_pad[:, :_NH], x_e2], axis=1))
    zeros = jnp.zeros((_NH, _NC), _F32)
    wa = jnp.concatenate([W3[:_NH], zeros, zeros], axis=1)
    wb = jnp.concatenate([W3[_NH:2 * _NH], W4, zeros], axis=1)
    wc3 = jnp.concatenate([W3[2 * _NH:], zeros, W4], axis=1)
    g = _t_pass(T, v, d, x1p, Wl1, bl1, Wl2, bl2, wa, wb, wc3)
    bcat = jnp.concatenate([b3, b4, b4])
    return _final_pass(nadj, g, bcat)


# speedup vs baseline: 1.5372x; 1.0262x over previous
import functools

import jax
import jax.numpy as jnp
from jax import lax
from jax.experimental import pallas as pl
from jax.experimental.pallas import tpu as pltpu
from jax.experimental.pallas import tpu_sc as plsc

_N = 4096
_E = 8192
_NH = 64
_NC = 16

_F32 = jnp.float32
_BF16 = jnp.bfloat16


def _bf(x):
    return x.astype(_BF16)


_BMN = 512
_BME = 256


def _pass1_body(xn_ref, wc_ref, a_ref, b1_ref, b2_ref, x1_ref, d_ref, s_ref):
    @pl.when(pl.program_id(0) == 0)
    def _():
        s_ref[...] = _bf(jnp.dot(xn_ref[...], wc_ref[...],
                                 preferred_element_type=_F32))

    y = jnp.dot(_bf(a_ref[...]), s_ref[...], preferred_element_type=_F32)
    x1 = y[:, :_NH] + b1_ref[...]
    x2 = y[:, _NH:] + b2_ref[...]
    x1_ref[...] = jnp.concatenate([x1, jnp.zeros_like(x1)], axis=1)
    d_ref[...] = x1 * x2 + 2.0 * x1


def _pass1(x_n, wc, nadj, b1, b2):
    return pl.pallas_call(
        _pass1_body,
        grid=(_N // _BMN,),
        in_specs=[
            pl.BlockSpec((_N, 2 * _NH), lambda i: (0, 0)),
            pl.BlockSpec((2 * _NH, 2 * _NH), lambda i: (0, 0)),
            pl.BlockSpec((_BMN, _N), lambda i: (i, 0)),
            pl.BlockSpec((1, _NH), lambda i: (0, 0)),
            pl.BlockSpec((1, _NH), lambda i: (0, 0)),
        ],
        out_specs=[
            pl.BlockSpec((_BMN, 2 * _NH), lambda i: (i, 0)),
            pl.BlockSpec((_BMN, _NH), lambda i: (i, 0)),
        ],
        out_shape=[jax.ShapeDtypeStruct((_N, 2 * _NH), _F32),
                   jax.ShapeDtypeStruct((_N, _NH), _F32)],
        scratch_shapes=[pltpu.VMEM((_N, 2 * _NH), _BF16)],
        compiler_params=pltpu.CompilerParams(
            dimension_semantics=("arbitrary",)),
    )(x_n, wc, nadj, b1.reshape(1, _NH), b2.reshape(1, _NH))


_SC_CORES = 2
_SC_SUBCORES = 16
_NW = _SC_CORES * _SC_SUBCORES
_EPW = _E // _NW
_CHUNK = 128
_NCHUNK = _EPW // _CHUNK


def _gather_body(x1_hbm, e0_hbm, e1_hbm, out_hbm, i0_v, i1_v, r0_v, r1_v, sem):
    wid = lax.axis_index("s") * _SC_CORES + lax.axis_index("c")
    row = wid * _NCHUNK
    pltpu.sync_copy(e0_hbm.at[pl.ds(row, _NCHUNK)], i0_v)
    pltpu.sync_copy(e1_hbm.at[pl.ds(row, _NCHUNK)], i1_v)
    copies = []
    for c in range(_NCHUNK):
        copies.append(pltpu.async_copy(
            x1_hbm.at[i0_v.at[c]], r0_v.at[pl.ds(c * _CHUNK, _CHUNK)], sem))
        copies.append(pltpu.async_copy(
            x1_hbm.at[i1_v.at[c]], r1_v.at[pl.ds(c * _CHUNK, _CHUNK)], sem))
    for cp in copies:
        cp.wait()

    def body(r, carry):
        for j in range(_NH // 16):
            a = r0_v[r, pl.ds(j * 16, 16)]
            b = r1_v[r, pl.ds(j * 16, 16)]
            r0_v[r, pl.ds(j * 16, 16)] = jnp.maximum(a * b, 0.0)
        return carry

    lax.fori_loop(0, _EPW, body, 0)
    pltpu.sync_copy(r0_v, out_hbm.at[pl.ds(wid * _EPW, _EPW)])


def _edge_gather(x1p, e0, e1):
    mesh = plsc.VectorSubcoreMesh(core_axis_name="c", subcore_axis_name="s")
    f = functools.partial(
        pl.kernel,
        mesh=mesh,
        out_type=jax.ShapeDtypeStruct((_E, 2 * _NH), _F32),
        scratch_types=[
            pltpu.VMEM((_NCHUNK, _CHUNK), jnp.int32),
            pltpu.VMEM((_NCHUNK, _CHUNK), jnp.int32),
            pltpu.VMEM((_EPW, 2 * _NH), _F32),
            pltpu.VMEM((_EPW, 2 * _NH), _F32),
            pltpu.SemaphoreType.DMA,
        ],
    )(_gather_body)
    return f(x1p, e0, e1)


def _eadj_body(xep_ref, w6p_ref, a_ref, b_ref, o_ref, u_ref):
    @pl.when(pl.program_id(0) == 0)
    def _():
        u_ref[...] = _bf(jnp.dot(xep_ref[...], w6p_ref[...],
                                 preferred_element_type=_F32))

    y = jnp.dot(_bf(a_ref[...]), u_ref[...], preferred_element_type=_F32)
    o_ref[...] = _bf(jnp.maximum(y + b_ref[...], 0.0))


def _eadj_pass(x_e_pad, w6p, eadj, b6):
    return pl.pallas_call(
        _eadj_body,
        grid=(_E // _BME,),
        in_specs=[
            pl.BlockSpec((_E, 2 * _NH), lambda i: (0, 0)),
            pl.BlockSpec((2 * _NH, _NH), lambda i: (0, 0)),
            pl.BlockSpec((_BME, _E), lambda i: (i, 0)),
            pl.BlockSpec((1, _NH), lambda i: (0, 0)),
        ],
        out_specs=pl.BlockSpec((_BME, _NH), lambda i: (i, 0)),
        out_shape=jax.ShapeDtypeStruct((_E, _NH), _BF16),
        scratch_shapes=[pltpu.VMEM((_E, _NH), _BF16)],
        compiler_params=pltpu.CompilerParams(
            dimension_semantics=("arbitrary",)),
    )(x_e_pad, w6p, eadj, b6.reshape(1, _NH))


def _tpass_body(t_ref, xep_ref, xe2_ref, d_ref, x1_ref, wl1_ref, bl1_ref,
                wl2_ref, bl2_ref, wa_ref, wb_ref, wc_ref, g_ref):
    t_bf = _bf(t_ref[...])
    y1 = jnp.dot(t_bf, _bf(xep_ref[:, :_NH]), preferred_element_type=_F32)
    y2 = jnp.dot(t_bf, xe2_ref[...], preferred_element_type=_F32)
    x3 = (jnp.dot(y1, wl1_ref[...], preferred_element_type=_F32)
          + bl1_ref[...] + d_ref[...])
    xe3 = (jnp.dot(y2, wl2_ref[...], preferred_element_type=_F32)
           + bl2_ref[...])
    x1 = x1_ref[:, :_NH]
    g_ref[...] = _bf(
        jnp.dot(x3, wa_ref[...], preferred_element_type=_F32)
        + jnp.dot(x1, wb_ref[...], preferred_element_type=_F32)
        + jnp.dot(xe3, wc_ref[...], preferred_element_type=_F32))


def _t_pass(t, x_e_pad, x_e2, d, x1p, wl1, bl1, wl2, bl2, wa, wb, wc):
    ng = 3 * _NC
    return pl.pallas_call(
        _tpass_body,
        grid=(_N // _BME,),
        in_specs=[
            pl.BlockSpec((_BME, _E), lambda i: (i, 0)),
            pl.BlockSpec((_E, 2 * _NH), lambda i: (0, 0)),
            pl.BlockSpec((_E, _NH), lambda i: (0, 0)),
            pl.BlockSpec((_BME, _NH), lambda i: (i, 0)),
            pl.BlockSpec((_BME, 2 * _NH), lambda i: (i, 0)),
            pl.BlockSpec((_NH, _NH), lambda i: (0, 0)),
            pl.BlockSpec((1, _NH), lambda i: (0, 0)),
            pl.BlockSpec((_NH, _NH), lambda i: (0, 0)),
            pl.BlockSpec((1, _NH), lambda i: (0, 0)),
            pl.BlockSpec((_NH, ng), lambda i: (0, 0)),
            pl.BlockSpec((_NH, ng), lambda i: (0, 0)),
            pl.BlockSpec((_NH, ng), lambda i: (0, 0)),
        ],
        out_specs=pl.BlockSpec((_BME, ng), lambda i: (i, 0)),
        out_shape=jax.ShapeDtypeStruct((_N, ng), _BF16),
        compiler_params=pltpu.CompilerParams(
            dimension_semantics=("arbitrary",)),
    )(t, x_e_pad, x_e2, d, x1p, wl1, bl1.reshape(1, _NH), wl2,
      bl2.reshape(1, _NH), wa, wb, wc)


def _final_body(a_ref, g_ref, b_ref, o1_ref, o2_ref, o3_ref):
    y = jnp.dot(_bf(a_ref[...]), g_ref[...], preferred_element_type=_F32)
    y = y + b_ref[...]
    for o_ref, lo in ((o1_ref, 0), (o2_ref, _NC), (o3_ref, 2 * _NC)):
        o = y[:, lo:lo + _NC]
        m = jnp.max(o, axis=1, keepdims=True)
        ls = jnp.log(jnp.sum(jnp.exp(o - m), axis=1, keepdims=True)) + m
        o_ref[...] = o - ls


def _final_pass(nadj, g, bcat):
    ng = 3 * _NC
    return pl.pallas_call(
        _final_body,
        grid=(_N // _BMN,),
        in_specs=[
            pl.BlockSpec((_BMN, _N), lambda i: (i, 0)),
            pl.BlockSpec((_N, ng), lambda i: (0, 0)),
            pl.BlockSpec((1, ng), lambda i: (0, 0)),
        ],
        out_specs=[pl.BlockSpec((_BMN, _NC), lambda i: (i, 0))] * 3,
        out_shape=[jax.ShapeDtypeStruct((_N, _NC), _F32)] * 3,
        compiler_params=pltpu.CompilerParams(
            dimension_semantics=("arbitrary",)),
    )(nadj, g, bcat.reshape(1, ng))


def kernel(X_n, nadj, edge_name, T, eadj, W1, b1, W2, b2, W6, b6, W3, b3,
           W4, b4, Wl1, bl1, Wl2, bl2):
    e0 = edge_name[:, 0].reshape(_E // _CHUNK, _CHUNK)
    e1 = edge_name[:, 1].reshape(_E // _CHUNK, _CHUNK)

    wc = jnp.concatenate([W1, W2], axis=1)
    x1p, d = _pass1(X_n, wc, nadj, b1, b2)
    x_e_pad = _edge_gather(x1p, e0, e1)
    w6p = jnp.concatenate([W6, jnp.zeros((_NH, _NH), _F32)], axis=0)
    x_e2 = _eadj_pass(x_e_pad, w6p, eadj, b6)
    zeros = jnp.zeros((_NH, _NC), _F32)
    wa = jnp.concatenate([W3[:_NH], zeros, zeros], axis=1)
    wb = jnp.concatenate([W3[_NH:2 * _NH], W4, zeros], axis=1)
    wc3 = jnp.concatenate([W3[2 * _NH:], zeros, W4], axis=1)
    g = _t_pass(T, x_e_pad, x_e2, d, x1p, Wl1, bl1, Wl2, bl2, wa, wb, wc3)
    bcat = jnp.concatenate([b3, b4, b4])
    return _final_pass(nadj, g, bcat)


# speedup vs baseline: 1.5399x; 1.0017x over previous
import functools

import jax
import jax.numpy as jnp
from jax import lax
from jax.experimental import pallas as pl
from jax.experimental.pallas import tpu as pltpu
from jax.experimental.pallas import tpu_sc as plsc

_N = 4096
_E = 8192
_NH = 64
_NC = 16

_F32 = jnp.float32
_BF16 = jnp.bfloat16


def _bf(x):
    return x.astype(_BF16)


_BMN = 512
_BME = 256


def _pass1_body(xn_ref, wc_ref, a0_ref, a1_ref, b1_ref, b2_ref, x1_ref,
                s_ref):
    @pl.when(pl.program_id(0) == 0)
    def _():
        s_ref[...] = _bf(jnp.dot(xn_ref[...], wc_ref[...],
                                 preferred_element_type=_F32))

    s = s_ref[...]
    y0 = jnp.dot(_bf(a0_ref[...]), s, preferred_element_type=_F32)
    y1 = jnp.dot(_bf(a1_ref[...]), s, preferred_element_type=_F32)
    y = jnp.concatenate([y0, y1], axis=0)
    x1 = y[:, :_NH] + b1_ref[...]
    x2 = y[:, _NH:] + b2_ref[...]
    x1_ref[...] = jnp.concatenate([x1, x2], axis=1)


def _pass1(x_n, wc, nadj, b1, b2):
    return pl.pallas_call(
        _pass1_body,
        grid=(_N // _BMN // 2,),
        in_specs=[
            pl.BlockSpec((_N, 2 * _NH), lambda i: (0, 0)),
            pl.BlockSpec((2 * _NH, 2 * _NH), lambda i: (0, 0)),
            pl.BlockSpec((_BMN, _N), lambda i: (2 * i, 0)),
            pl.BlockSpec((_BMN, _N), lambda i: (2 * i + 1, 0)),
            pl.BlockSpec((1, _NH), lambda i: (0, 0)),
            pl.BlockSpec((1, _NH), lambda i: (0, 0)),
        ],
        out_specs=pl.BlockSpec((2 * _BMN, 2 * _NH), lambda i: (i, 0)),
        out_shape=jax.ShapeDtypeStruct((_N, 2 * _NH), _F32),
        scratch_shapes=[pltpu.VMEM((_N, 2 * _NH), _BF16)],
        compiler_params=pltpu.CompilerParams(
            dimension_semantics=("arbitrary",)),
    )(x_n, wc, nadj, nadj, b1.reshape(1, _NH), b2.reshape(1, _NH))


_SC_CORES = 2
_SC_SUBCORES = 16
_NW = _SC_CORES * _SC_SUBCORES
_EPW = _E // _NW
_CHUNK = 128
_NCHUNK = _EPW // _CHUNK


def _gather_body(x1_hbm, e0_hbm, e1_hbm, out_hbm, i0_v, i1_v, r0_v, r1_v, sem):
    wid = lax.axis_index("s") * _SC_CORES + lax.axis_index("c")
    row = wid * _NCHUNK
    pltpu.sync_copy(e0_hbm.at[pl.ds(row, _NCHUNK)], i0_v)
    pltpu.sync_copy(e1_hbm.at[pl.ds(row, _NCHUNK)], i1_v)
    copies = []
    for c in range(_NCHUNK):
        copies.append(pltpu.async_copy(
            x1_hbm.at[i0_v.at[c]], r0_v.at[pl.ds(c * _CHUNK, _CHUNK)], sem))
        copies.append(pltpu.async_copy(
            x1_hbm.at[i1_v.at[c]], r1_v.at[pl.ds(c * _CHUNK, _CHUNK)], sem))
    for cp in copies:
        cp.wait()

    def body(r, carry):
        for j in range(_NH // 16):
            a = r0_v[r, pl.ds(j * 16, 16)]
            b = r1_v[r, pl.ds(j * 16, 16)]
            r0_v[r, pl.ds(j * 16, 16)] = jnp.maximum(a * b, 0.0)
        return carry

    lax.fori_loop(0, _EPW, body, 0)
    pltpu.sync_copy(r0_v, out_hbm.at[pl.ds(wid * _EPW, _EPW)])


def _edge_gather(x1p, e0, e1):
    mesh = plsc.VectorSubcoreMesh(core_axis_name="c", subcore_axis_name="s")
    f = functools.partial(
        pl.kernel,
        mesh=mesh,
        out_type=jax.ShapeDtypeStruct((_E, 2 * _NH), _F32),
        scratch_types=[
            pltpu.VMEM((_NCHUNK, _CHUNK), jnp.int32),
            pltpu.VMEM((_NCHUNK, _CHUNK), jnp.int32),
            pltpu.VMEM((_EPW, 2 * _NH), _F32),
            pltpu.VMEM((_EPW, 2 * _NH), _F32),
            pltpu.SemaphoreType.DMA,
        ],
    )(_gather_body)
    return f(x1p, e0, e1)


def _eadj_body(xep_ref, w6p_ref, a_ref, b_ref, o_ref, u_ref):
    @pl.when(pl.program_id(0) == 0)
    def _():
        u_ref[...] = _bf(jnp.dot(xep_ref[...], w6p_ref[...],
                                 preferred_element_type=_F32))

    y = jnp.dot(_bf(a_ref[...]), u_ref[...], preferred_element_type=_F32)
    o_ref[...] = _bf(jnp.maximum(y + b_ref[...], 0.0))


def _eadj_pass(x_e_pad, w6p, eadj, b6):
    return pl.pallas_call(
        _eadj_body,
        grid=(_E // _BME,),
        in_specs=[
            pl.BlockSpec((_E, 2 * _NH), lambda i: (0, 0)),
            pl.BlockSpec((2 * _NH, _NH), lambda i: (0, 0)),
            pl.BlockSpec((_BME, _E), lambda i: (i, 0)),
            pl.BlockSpec((1, _NH), lambda i: (0, 0)),
        ],
        out_specs=pl.BlockSpec((_BME, _NH), lambda i: (i, 0)),
        out_shape=jax.ShapeDtypeStruct((_E, _NH), _BF16),
        scratch_shapes=[pltpu.VMEM((_E, _NH), _BF16)],
        compiler_params=pltpu.CompilerParams(
            dimension_semantics=("arbitrary",)),
    )(x_e_pad, w6p, eadj, b6.reshape(1, _NH))


def _tpass_body(t_ref, xep_ref, xe2_ref, x1_ref, wl1_ref, bl1_ref,
                wl2_ref, bl2_ref, wa_ref, wb_ref, wc_ref, g_ref):
    t_bf = _bf(t_ref[...])
    y1 = jnp.dot(t_bf, _bf(xep_ref[:, :_NH]), preferred_element_type=_F32)
    y2 = jnp.dot(t_bf, xe2_ref[...], preferred_element_type=_F32)
    x1 = x1_ref[:, :_NH]
    x2 = x1_ref[:, _NH:]
    d = x1 * x2 + 2.0 * x1
    x3 = (jnp.dot(y1, wl1_ref[...], preferred_element_type=_F32)
          + bl1_ref[...] + d)
    xe3 = (jnp.dot(y2, wl2_ref[...], preferred_element_type=_F32)
           + bl2_ref[...])
    g_ref[...] = _bf(
        jnp.dot(x3, wa_ref[...], preferred_element_type=_F32)
        + jnp.dot(x1, wb_ref[...], preferred_element_type=_F32)
        + jnp.dot(xe3, wc_ref[...], preferred_element_type=_F32))


def _t_pass(t, x_e_pad, x_e2, x1p, wl1, bl1, wl2, bl2, wa, wb, wc):
    ng = 3 * _NC
    return pl.pallas_call(
        _tpass_body,
        grid=(_N // _BME,),
        in_specs=[
            pl.BlockSpec((_BME, _E), lambda i: (i, 0)),
            pl.BlockSpec((_E, 2 * _NH), lambda i: (0, 0)),
            pl.BlockSpec((_E, _NH), lambda i: (0, 0)),
            pl.BlockSpec((_BME, 2 * _NH), lambda i: (i, 0)),
            pl.BlockSpec((_NH, _NH), lambda i: (0, 0)),
            pl.BlockSpec((1, _NH), lambda i: (0, 0)),
            pl.BlockSpec((_NH, _NH), lambda i: (0, 0)),
            pl.BlockSpec((1, _NH), lambda i: (0, 0)),
            pl.BlockSpec((_NH, ng), lambda i: (0, 0)),
            pl.BlockSpec((_NH, ng), lambda i: (0, 0)),
            pl.BlockSpec((_NH, ng), lambda i: (0, 0)),
        ],
        out_specs=pl.BlockSpec((_BME, ng), lambda i: (i, 0)),
        out_shape=jax.ShapeDtypeStruct((_N, ng), _BF16),
        compiler_params=pltpu.CompilerParams(
            dimension_semantics=("arbitrary",)),
    )(t, x_e_pad, x_e2, x1p, wl1, bl1.reshape(1, _NH), wl2,
      bl2.reshape(1, _NH), wa, wb, wc)


def _final_body(a0_ref, a1_ref, a2_ref, a3_ref, g_ref, b_ref,
                o1_ref, o2_ref, o3_ref):
    g = g_ref[...]
    ys = [jnp.dot(_bf(a_ref[...]), g, preferred_element_type=_F32)
          for a_ref in (a0_ref, a1_ref, a2_ref, a3_ref)]
    y = jnp.concatenate(ys, axis=0) + b_ref[...]
    for o_ref, lo in ((o1_ref, 0), (o2_ref, _NC), (o3_ref, 2 * _NC)):
        o = y[:, lo:lo + _NC]
        m = jnp.max(o, axis=1, keepdims=True)
        ls = jnp.log(jnp.sum(jnp.exp(o - m), axis=1, keepdims=True)) + m
        o_ref[...] = o - ls


_BMF = 256


def _final_pass(nadj, g, bcat):
    ng = 3 * _NC
    return pl.pallas_call(
        _final_body,
        grid=(_N // _BMF // 4,),
        in_specs=[
            pl.BlockSpec((_BMF, _N), lambda i: (4 * i, 0)),
            pl.BlockSpec((_BMF, _N), lambda i: (4 * i + 1, 0)),
            pl.BlockSpec((_BMF, _N), lambda i: (4 * i + 2, 0)),
            pl.BlockSpec((_BMF, _N), lambda i: (4 * i + 3, 0)),
            pl.BlockSpec((_N, ng), lambda i: (0, 0)),
            pl.BlockSpec((1, ng), lambda i: (0, 0)),
        ],
        out_specs=[pl.BlockSpec((4 * _BMF, _NC), lambda i: (i, 0))] * 3,
        out_shape=[jax.ShapeDtypeStruct((_N, _NC), _F32)] * 3,
        compiler_params=pltpu.CompilerParams(
            dimension_semantics=("arbitrary",)),
    )(nadj, nadj, nadj, nadj, g, bcat.reshape(1, ng))


def kernel(X_n, nadj, edge_name, T, eadj, W1, b1, W2, b2, W6, b6, W3, b3,
           W4, b4, Wl1, bl1, Wl2, bl2):
    e0 = edge_name[:, 0].reshape(_E // _CHUNK, _CHUNK)
    e1 = edge_name[:, 1].reshape(_E // _CHUNK, _CHUNK)

    wc = jnp.concatenate([W1, W2], axis=1)
    x1p = _pass1(X_n, wc, nadj, b1, b2)
    x_e_pad = _edge_gather(x1p, e0, e1)
    w6p = jnp.concatenate([W6, jnp.zeros((_NH, _NH), _F32)], axis=0)
    x_e2 = _eadj_pass(x_e_pad, w6p, eadj, b6)
    zeros = jnp.zeros((_NH, _NC), _F32)
    wa = jnp.concatenate([W3[:_NH], zeros, zeros], axis=1)
    wb = jnp.concatenate([W3[_NH:2 * _NH], W4, zeros], axis=1)
    wc3 = jnp.concatenate([W3[2 * _NH:], zeros, W4], axis=1)
    g = _t_pass(T, x_e_pad, x_e2, x1p, Wl1, bl1, Wl2, bl2, wa, wb, wc3)
    bcat = jnp.concatenate([b3, b4, b4])
    return _final_pass(nadj, g, bcat)
